# Initial kernel scaffold; baseline (speedup 1.0000x reference)
import jax
import jax.numpy as jnp
from jax.experimental import pallas as pl


def kernel(idx, targets, table):
    raise NotImplementedError("write your pallas kernel here")


# speedup vs baseline: 1.2512x; 1.2512x over previous
import functools

import jax
import jax.numpy as jnp
from jax import lax
from jax.experimental import pallas as pl
from jax.experimental.pallas import tpu as pltpu
from jax.experimental.pallas import tpu_sc as plsc

_LANES = 16
_CHUNK = 32


def _lse_body(table_ref, lse_ref):
    t = table_ref[...]
    m = jnp.max(t, axis=1, keepdims=True)
    s = jnp.sum(jnp.exp(t - m), axis=1, keepdims=True)
    lse_ref[...] = jnp.log(s) + m


def _row_lse(table):
    v = table.shape[0]
    return pl.pallas_call(
        _lse_body,
        out_shape=jax.ShapeDtypeStruct((v, 1), jnp.float32),
    )(table)


@functools.lru_cache(maxsize=None)
def _make_sc_kernel(nt, v, d, nc, ns):
    nw = nc * ns
    per_w = nt // nw
    assert per_w * nw == nt
    n_chunks = per_w // _CHUNK
    assert n_chunks * _CHUNK == per_w and n_chunks % 2 == 0

    mesh = plsc.VectorSubcoreMesh(core_axis_name="c", subcore_axis_name="s")

    @functools.partial(
        pl.kernel,
        mesh=mesh,
        compiler_params=pltpu.CompilerParams(use_tc_tiling_on_sc=False),
        out_type=[
            jax.ShapeDtypeStruct((nt, d), jnp.float32),
            jax.ShapeDtypeStruct((nw, _LANES), jnp.float32),
        ],
        scratch_types=[
            pltpu.VMEM((per_w,), jnp.int32),
            pltpu.VMEM((per_w,), jnp.int32),
            pltpu.VMEM((per_w,), jnp.float32),
            pltpu.VMEM((per_w,), jnp.float32),
            pltpu.VMEM((_CHUNK, d), jnp.float32),
            pltpu.VMEM((_CHUNK, d), jnp.float32),
            pltpu.VMEM((_LANES,), jnp.float32),
            pltpu.SemaphoreType.DMA,
            pltpu.SemaphoreType.DMA,
            pltpu.SemaphoreType.DMA,
            pltpu.SemaphoreType.DMA,
            pltpu.SemaphoreType.DMA,
        ],
    )
    def sc_kernel(table_hbm, cat_hbm, idx_hbm, fidx_hbm,
                  out_hbm, part_hbm,
                  idx_v, fidx_v, picked_v, lsetok_v, buf0, buf1, acc_v,
                  gsem0, gsem1, osem0, osem1, psem):
        wid = lax.axis_index("s") * nc + lax.axis_index("c")
        base = wid * per_w
        pltpu.sync_copy(idx_hbm.at[pl.ds(base, per_w)], idx_v)
        pltpu.sync_copy(fidx_hbm.at[pl.ds(base, per_w)], fidx_v)
        pick_dma = pltpu.make_async_copy(
            cat_hbm.at[fidx_v], picked_v, psem)
        pick_dma.start()
        lse_dma = pltpu.make_async_copy(
            cat_hbm.at[idx_v], lsetok_v, psem)
        lse_dma.start()

        bufs = (buf0, buf1)
        gsems = (gsem0, gsem1)
        osems = (osem0, osem1)

        def gather(g, b):
            idx_slice = idx_v.at[pl.ds(g * _CHUNK, _CHUNK)]
            return pltpu.make_async_copy(
                table_hbm.at[idx_slice], bufs[b], gsems[b])

        def outcopy(g, b):
            dst = out_hbm.at[pl.ds(base + g * _CHUNK, _CHUNK)]
            return pltpu.make_async_copy(bufs[b], dst, osems[b])

        gather(0, 0).start()
        gather(1, 1).start()

        def pair_body(p, carry):
            for b in range(2):
                g = 2 * p + b
                gather(g, b).wait()
                outcopy(g, b).start()

                @pl.when(g + 2 < n_chunks)
                def _():
                    outcopy(g, b).wait()
                    gather(g + 2, b).start()
            return carry

        lax.fori_loop(0, n_chunks // 2, pair_body, 0)
        outcopy(n_chunks - 2, 0).wait()
        outcopy(n_chunks - 1, 1).wait()

        pick_dma.wait()
        lse_dma.wait()
        acc_v[...] = jnp.zeros((_LANES,), jnp.float32)

        def loss_body(i, carry):
            o = i * _LANES
            acc_v[...] = acc_v[...] + (
                lsetok_v[pl.ds(o, _LANES)] - picked_v[pl.ds(o, _LANES)])
            return carry

        lax.fori_loop(0, per_w // _LANES, loss_body, 0)
        pltpu.sync_copy(acc_v, part_hbm.at[wid])

    return sc_kernel


def kernel(idx, targets, table):
    b, t = idx.shape
    v, d = table.shape
    nt = b * t
    lse = _row_lse(table).reshape(v)
    idx_f = idx.reshape(nt).astype(jnp.int32)
    fidx = v + idx_f * d + targets.reshape(nt).astype(jnp.int32)
    cat = jnp.concatenate([lse, table.reshape(v * d)])
    info = plsc.get_sparse_core_info()
    sck = _make_sc_kernel(nt, v, d, info.num_cores, info.num_subcores)
    logits_flat, parts = sck(table, cat, idx_f, fidx)
    loss = jnp.sum(parts) / nt
    return logits_flat.reshape(b, t, v), loss
